# Initial kernel scaffold; baseline (speedup 1.0000x reference)
import jax
import jax.numpy as jnp
from jax.experimental import pallas as pl


def kernel(news_x, company_x, edge_index, num_companies, W_news, W_company, v):
    raise NotImplementedError("write your pallas kernel here")


# speedup vs baseline: 4.9694x; 4.9694x over previous
import functools

import jax
import jax.numpy as jnp
from jax import lax
from jax.experimental import pallas as pl
from jax.experimental.pallas import tpu as pltpu
from jax.experimental.pallas import tpu_sc as plsc

NC, NS, L = 2, 16, 16
NW = NC * NS
H = 128
HB = H // L
CHUNK = 80

CP = 10240
DEN0 = CP
TR = 10368


def _proj_body(nx_ref, cx_ref, wn_ref, wc_ref, a_ref, b_ref):
    dn = (((1,), (1,)), ((), ()))
    a_ref[...] = lax.dot_general(nx_ref[...], wn_ref[...], dn,
                                 preferred_element_type=jnp.float32)
    b_ref[...] = lax.dot_general(cx_ref[...], wc_ref[...], dn,
                                 preferred_element_type=jnp.float32)


def _score_body(g_ref, v_ref, p_ref):
    t = jnp.tanh(g_ref[...])
    s = jnp.sum(t * v_ref[...], axis=1)
    p_ref[...] = jnp.exp(s)


def _final_body(f0_ref, f1_ref, d0_ref, d1_ref, o_ref):
    acc = f0_ref[0] + f1_ref[0]
    dpk = d0_ref[0] + d1_ref[0]
    n = acc.shape[0]
    rsel = (lax.broadcasted_iota(jnp.int32, (n, n // H), 0) // H
            == lax.broadcasted_iota(jnp.int32, (n, n // H), 1))
    den_rows = jax.lax.dot_general(rsel.astype(jnp.float32), dpk,
                                   (((1,), (0,)), ((), ())),
                                   preferred_element_type=jnp.float32)
    lsel = (lax.broadcasted_iota(jnp.int32, (n, H), 1)
            == lax.broadcasted_iota(jnp.int32, (n, H), 0) % H)
    den = jnp.sum(jnp.where(lsel, den_rows, 0.0), axis=1, keepdims=True)
    o_ref[...] = acc / jnp.maximum(den, 1e-9)


def _gather_add_body(src_hbm, dst_hbm, a_hbm, b_hbm, g_hbm,
                     si_v, di_v, g_v, sem):
    wid = lax.axis_index("s") * NC + lax.axis_index("c")
    epw = src_hbm.shape[0] // NW
    base = wid * epw

    def step(j, _):
        off = base + j * CHUNK
        pltpu.sync_copy(src_hbm.at[pl.ds(off, CHUNK)], si_v)
        pltpu.sync_copy(dst_hbm.at[pl.ds(off, CHUNK)], di_v)
        pltpu.async_copy(a_hbm.at[si_v], g_v, sem).wait()
        pltpu.async_copy(b_hbm.at[di_v], g_v, sem, add=True).wait()
        pltpu.sync_copy(g_v, g_hbm.at[pl.ds(off, CHUNK)])
        return 0

    lax.fori_loop(0, epw // CHUNK, step, 0)


def _accum_body(src_hbm, dst_hbm, x_hbm, p_hbm, z_hbm, parts_hbm,
                si_v, di_v, d2_v, x_v, p_v, row_v, oh_v, acc_sh, sem):
    cid = lax.axis_index("c")
    sid = lax.axis_index("s")
    wid = sid * NC + cid
    epw = src_hbm.shape[0] // NW
    base = wid * epw
    rpt = acc_sh.shape[0] // NS

    pltpu.sync_copy(z_hbm, acc_sh.at[pl.ds(sid * rpt, rpt)])
    plsc.subcore_barrier()

    lane_iota = [lax.iota(jnp.int32, L) + h * L for h in range(HB)]

    def edge_group(k, _):
        ps = p_v[pl.ds(k * L, L)]
        di = di_v[pl.ds(k * L, L)]
        dm = jnp.bitwise_and(di, H - 1)
        for j in range(L):
            e = k * L + j
            pb = lax.broadcast(ps[j], (L,))
            db = lax.broadcast(dm[j], (L,))
            for h in range(HB):
                row_v[e, pl.ds(h * L, L)] = x_v[e, pl.ds(h * L, L)] * pb
                oh_v[e, pl.ds(h * L, L)] = jnp.where(lane_iota[h] == db, pb,
                                                     0.0)
        return 0

    def step(j, _):
        off = base + j * CHUNK
        pltpu.sync_copy(src_hbm.at[pl.ds(off, CHUNK)], si_v)
        pltpu.sync_copy(dst_hbm.at[pl.ds(off, CHUNK)], di_v)
        pltpu.sync_copy(p_hbm.at[pl.ds(off, CHUNK)], p_v)
        pltpu.async_copy(x_hbm.at[si_v], x_v, sem).wait()

        def mk_d2(k, _):
            di = di_v[pl.ds(k * L, L)]
            d2_v[pl.ds(k * L, L)] = DEN0 + lax.shift_right_logical(di, 7)
            return 0

        lax.fori_loop(0, CHUNK // L, mk_d2, 0)
        lax.fori_loop(0, CHUNK // L, edge_group, 0)
        pltpu.sync_copy(row_v, acc_sh.at[di_v], add=True)
        pltpu.sync_copy(oh_v, acc_sh.at[d2_v], add=True)
        return 0

    lax.fori_loop(0, epw // CHUNK, step, 0)

    plsc.subcore_barrier()
    pltpu.sync_copy(acc_sh.at[pl.ds(sid * rpt, rpt)],
                    parts_hbm.at[cid, pl.ds(sid * rpt, rpt)])


def kernel(news_x, company_x, edge_index, num_companies, W_news, W_company, v):
    N, Hd = news_x.shape
    C = company_x.shape[0]
    E = edge_index.shape[1]
    src = edge_index[0]
    dst = edge_index[1]

    rows_blk = 1000
    a, b = pl.pallas_call(
        _proj_body,
        grid=(N // rows_blk,),
        in_specs=[
            pl.BlockSpec((rows_blk, Hd), lambda i: (i, 0)),
            pl.BlockSpec((rows_blk, Hd), lambda i: (i, 0)),
            pl.BlockSpec((Hd, Hd), lambda i: (0, 0)),
            pl.BlockSpec((Hd, Hd), lambda i: (0, 0)),
        ],
        out_specs=[
            pl.BlockSpec((rows_blk, Hd), lambda i: (i, 0)),
            pl.BlockSpec((rows_blk, Hd), lambda i: (i, 0)),
        ],
        out_shape=[
            jax.ShapeDtypeStruct((N, Hd), jnp.float32),
            jax.ShapeDtypeStruct((C, Hd), jnp.float32),
        ],
    )(news_x, company_x, W_news, W_company)

    mesh = plsc.VectorSubcoreMesh(core_axis_name="c", subcore_axis_name="s",
                                  num_cores=NC, num_subcores=NS)

    g = pl.kernel(
        _gather_add_body,
        out_type=jax.ShapeDtypeStruct((E, Hd), jnp.float32),
        mesh=mesh,
        scratch_types=[
            pltpu.VMEM((CHUNK,), jnp.int32),
            pltpu.VMEM((CHUNK,), jnp.int32),
            pltpu.VMEM((CHUNK, Hd), jnp.float32),
            pltpu.SemaphoreType.DMA,
        ],
    )(src, dst, a, b)

    e_blk = 512
    p = pl.pallas_call(
        _score_body,
        grid=(E // e_blk,),
        in_specs=[
            pl.BlockSpec((e_blk, Hd), lambda i: (i, 0)),
            pl.BlockSpec((1, Hd), lambda i: (0, 0)),
        ],
        out_specs=pl.BlockSpec((e_blk,), lambda i: (i,)),
        out_shape=jax.ShapeDtypeStruct((E,), jnp.float32),
    )(g, v)

    zeros_tile = jnp.zeros((TR // NS, Hd), jnp.float32)
    parts = pl.kernel(
        _accum_body,
        out_type=jax.ShapeDtypeStruct((NC, TR, Hd), jnp.float32),
        mesh=mesh,
        scratch_types=[
            pltpu.VMEM((CHUNK,), jnp.int32),
            pltpu.VMEM((CHUNK,), jnp.int32),
            pltpu.VMEM((CHUNK,), jnp.int32),
            pltpu.VMEM((CHUNK, Hd), jnp.float32),
            pltpu.VMEM((CHUNK,), jnp.float32),
            pltpu.VMEM((CHUNK, Hd), jnp.float32),
            pltpu.VMEM((CHUNK, Hd), jnp.float32),
            pltpu.VMEM_SHARED((TR, Hd), jnp.float32),
            pltpu.SemaphoreType.DMA,
        ],
    )(src, dst, news_x, p, zeros_tile)

    c_blk = 1024
    n_blk = (C + c_blk - 1) // c_blk
    out = pl.pallas_call(
        _final_body,
        grid=(n_blk,),
        in_specs=[
            pl.BlockSpec((1, c_blk, Hd), lambda i: (0, i, 0)),
            pl.BlockSpec((1, c_blk, Hd), lambda i: (1, i, 0)),
            pl.BlockSpec((1, c_blk // Hd, Hd),
                         lambda i: (0, DEN0 // (c_blk // Hd) + i, 0)),
            pl.BlockSpec((1, c_blk // Hd, Hd),
                         lambda i: (1, DEN0 // (c_blk // Hd) + i, 0)),
        ],
        out_specs=pl.BlockSpec((c_blk, Hd), lambda i: (i, 0)),
        out_shape=jax.ShapeDtypeStruct((C, Hd), jnp.float32),
    )(parts, parts, parts, parts)
    return out


# speedup vs baseline: 7.6622x; 1.5419x over previous
import functools

import jax
import jax.numpy as jnp
from jax import lax
from jax.experimental import pallas as pl
from jax.experimental.pallas import tpu as pltpu
from jax.experimental.pallas import tpu_sc as plsc

NC, NS, L = 2, 16, 16
NW = NC * NS
H = 128
HB = H // L
CHUNK = 80

CP = 10240
DEN0 = CP
TR = 10368


def _proj_body(nx_ref, cx_ref, wn_ref, wc_ref, a_ref, b_ref):
    dn = (((1,), (1,)), ((), ()))
    a_ref[...] = lax.dot_general(nx_ref[...], wn_ref[...], dn,
                                 preferred_element_type=jnp.float32)
    b_ref[...] = lax.dot_general(cx_ref[...], wc_ref[...], dn,
                                 preferred_element_type=jnp.float32)


def _score_body(g_ref, v_ref, p_ref):
    t = jnp.tanh(g_ref[...])
    s = jnp.sum(t * v_ref[...], axis=1)
    p_ref[...] = jnp.exp(s)


def _final_body(f0_ref, f1_ref, d0_ref, d1_ref, o_ref):
    acc = f0_ref[0] + f1_ref[0]
    dpk = d0_ref[0] + d1_ref[0]
    n = acc.shape[0]
    rsel = (lax.broadcasted_iota(jnp.int32, (n, n // H), 0) // H
            == lax.broadcasted_iota(jnp.int32, (n, n // H), 1))
    den_rows = jax.lax.dot_general(rsel.astype(jnp.float32), dpk,
                                   (((1,), (0,)), ((), ())),
                                   preferred_element_type=jnp.float32)
    lsel = (lax.broadcasted_iota(jnp.int32, (n, H), 1)
            == lax.broadcasted_iota(jnp.int32, (n, H), 0) % H)
    den = jnp.sum(jnp.where(lsel, den_rows, 0.0), axis=1, keepdims=True)
    o_ref[...] = acc / jnp.maximum(den, 1e-9)


GC = 128


def _gather_add_body(src_hbm, dst_hbm, a_hbm, b_hbm, g_hbm,
                     si_v, di_v, g0_v, g1_v,
                     sa0, sa1, sb0, sb1, sw0, sw1):
    wid = lax.axis_index("s") * NC + lax.axis_index("c")
    epw = src_hbm.shape[0] // NW
    base = wid * epw
    n_full = epw // GC
    tail = epw - n_full * GC

    pltpu.sync_copy(src_hbm.at[pl.ds(base, epw)], si_v)
    pltpu.sync_copy(dst_hbm.at[pl.ds(base, epw)], di_v)

    def pair(k, _):
        c0 = 2 * k * GC
        c1 = c0 + GC
        a0 = pltpu.async_copy(a_hbm.at[si_v.at[pl.ds(c0, GC)]], g0_v, sa0)
        a1 = pltpu.async_copy(a_hbm.at[si_v.at[pl.ds(c1, GC)]], g1_v, sa1)
        a0.wait()
        b0 = pltpu.async_copy(b_hbm.at[di_v.at[pl.ds(c0, GC)]], g0_v, sb0,
                              add=True)
        a1.wait()
        b1 = pltpu.async_copy(b_hbm.at[di_v.at[pl.ds(c1, GC)]], g1_v, sb1,
                              add=True)
        b0.wait()
        w0 = pltpu.async_copy(g0_v, g_hbm.at[pl.ds(base + c0, GC)], sw0)
        b1.wait()
        w1 = pltpu.async_copy(g1_v, g_hbm.at[pl.ds(base + c1, GC)], sw1)
        w0.wait()
        w1.wait()
        return 0

    lax.fori_loop(0, n_full // 2, pair, 0)

    t0 = n_full * GC
    pltpu.async_copy(a_hbm.at[si_v.at[pl.ds(t0, tail)]],
                     g0_v.at[pl.ds(0, tail)], sa0).wait()
    pltpu.async_copy(b_hbm.at[di_v.at[pl.ds(t0, tail)]],
                     g0_v.at[pl.ds(0, tail)], sb0, add=True).wait()
    pltpu.async_copy(g0_v.at[pl.ds(0, tail)],
                     g_hbm.at[pl.ds(base + t0, tail)], sw0).wait()


SEC = 2000


def _accum_body(src_hbm, dst_hbm, x_hbm, p_hbm, z_hbm, parts_hbm,
                si_v, di_v, p_v, x0_v, x1_v, o0_v, o1_v,
                i0_v, i1_v, j0_v, j1_v, acc_sh,
                sg0, sg1, sf0, sf1, so0, so1):
    cid = lax.axis_index("c")
    sid = lax.axis_index("s")
    wid = sid * NC + cid
    epw = src_hbm.shape[0] // NW
    base = wid * epw
    rpt = acc_sh.shape[0] // NS

    pltpu.sync_copy(z_hbm, acc_sh.at[pl.ds(sid * rpt, rpt)])
    plsc.subcore_barrier()

    lane_iota = [lax.iota(jnp.int32, L) + h * L for h in range(HB)]

    def chunk_scatter(coff, x_v, o_v, i_v, j_v, sf, so):
        def group(k, _):
            eoff = coff + k * L
            ps = p_v[pl.ds(eoff, L)]
            di = di_v[pl.ds(eoff, L)]
            i_v[pl.ds(k * L, L)] = di
            j_v[pl.ds(k * L, L)] = DEN0 + lax.shift_right_logical(di, 7)
            dm = jnp.bitwise_and(di, H - 1)
            for j in range(L):
                e = k * L + j
                pb = lax.broadcast(ps[j], (L,))
                db = lax.broadcast(dm[j], (L,))
                for h in range(HB):
                    x_v[e, pl.ds(h * L, L)] = x_v[e, pl.ds(h * L, L)] * pb
                    o_v[e, pl.ds(h * L, L)] = jnp.where(lane_iota[h] == db,
                                                        pb, 0.0)
            return 0

        lax.fori_loop(0, CHUNK // L, group, 0)
        f = pltpu.async_copy(x_v, acc_sh.at[i_v], sf, add=True)
        d = pltpu.async_copy(o_v, acc_sh.at[j_v], so, add=True)
        return f, d

    def section(s, _):
        soff = base + s * SEC
        pltpu.sync_copy(src_hbm.at[pl.ds(soff, SEC)], si_v)
        pltpu.sync_copy(dst_hbm.at[pl.ds(soff, SEC)], di_v)
        pltpu.sync_copy(p_hbm.at[pl.ds(soff, SEC)], p_v)

        def pair(k, _):
            c0 = 2 * k * CHUNK
            c1 = c0 + CHUNK
            g0 = pltpu.async_copy(x_hbm.at[si_v.at[pl.ds(c0, CHUNK)]],
                                  x0_v, sg0)
            g1 = pltpu.async_copy(x_hbm.at[si_v.at[pl.ds(c1, CHUNK)]],
                                  x1_v, sg1)
            g0.wait()
            f0, d0 = chunk_scatter(c0, x0_v, o0_v, i0_v, j0_v, sf0, so0)
            g1.wait()
            f1, d1 = chunk_scatter(c1, x1_v, o1_v, i1_v, j1_v, sf1, so1)
            f0.wait()
            d0.wait()
            f1.wait()
            d1.wait()
            return 0

        n_chunks = SEC // CHUNK
        lax.fori_loop(0, n_chunks // 2, pair, 0)

        ct = (n_chunks - 1) * CHUNK
        pltpu.async_copy(x_hbm.at[si_v.at[pl.ds(ct, CHUNK)]],
                         x0_v, sg0).wait()
        ft, dt = chunk_scatter(ct, x0_v, o0_v, i0_v, j0_v, sf0, so0)
        ft.wait()
        dt.wait()
        return 0

    lax.fori_loop(0, epw // SEC, section, 0)

    plsc.subcore_barrier()
    pltpu.sync_copy(acc_sh.at[pl.ds(sid * rpt, rpt)],
                    parts_hbm.at[cid, pl.ds(sid * rpt, rpt)])


def kernel(news_x, company_x, edge_index, num_companies, W_news, W_company, v):
    N, Hd = news_x.shape
    C = company_x.shape[0]
    E = edge_index.shape[1]
    src = edge_index[0]
    dst = edge_index[1]

    rows_blk = 1000
    a, b = pl.pallas_call(
        _proj_body,
        grid=(N // rows_blk,),
        in_specs=[
            pl.BlockSpec((rows_blk, Hd), lambda i: (i, 0)),
            pl.BlockSpec((rows_blk, Hd), lambda i: (i, 0)),
            pl.BlockSpec((Hd, Hd), lambda i: (0, 0)),
            pl.BlockSpec((Hd, Hd), lambda i: (0, 0)),
        ],
        out_specs=[
            pl.BlockSpec((rows_blk, Hd), lambda i: (i, 0)),
            pl.BlockSpec((rows_blk, Hd), lambda i: (i, 0)),
        ],
        out_shape=[
            jax.ShapeDtypeStruct((N, Hd), jnp.float32),
            jax.ShapeDtypeStruct((C, Hd), jnp.float32),
        ],
    )(news_x, company_x, W_news, W_company)

    mesh = plsc.VectorSubcoreMesh(core_axis_name="c", subcore_axis_name="s",
                                  num_cores=NC, num_subcores=NS)

    g = pl.kernel(
        _gather_add_body,
        out_type=jax.ShapeDtypeStruct((E, Hd), jnp.float32),
        mesh=mesh,
        scratch_types=[
            pltpu.VMEM((E // NW,), jnp.int32),
            pltpu.VMEM((E // NW,), jnp.int32),
            pltpu.VMEM((GC, Hd), jnp.float32),
            pltpu.VMEM((GC, Hd), jnp.float32),
        ] + [pltpu.SemaphoreType.DMA] * 6,
    )(src, dst, a, b)

    e_blk = 512
    p = pl.pallas_call(
        _score_body,
        grid=(E // e_blk,),
        in_specs=[
            pl.BlockSpec((e_blk, Hd), lambda i: (i, 0)),
            pl.BlockSpec((1, Hd), lambda i: (0, 0)),
        ],
        out_specs=pl.BlockSpec((e_blk,), lambda i: (i,)),
        out_shape=jax.ShapeDtypeStruct((E,), jnp.float32),
    )(g, v)

    zeros_tile = jnp.zeros((TR // NS, Hd), jnp.float32)
    parts = pl.kernel(
        _accum_body,
        out_type=jax.ShapeDtypeStruct((NC, TR, Hd), jnp.float32),
        mesh=mesh,
        scratch_types=[
            pltpu.VMEM((SEC,), jnp.int32),
            pltpu.VMEM((SEC,), jnp.int32),
            pltpu.VMEM((SEC,), jnp.float32),
            pltpu.VMEM((CHUNK, Hd), jnp.float32),
            pltpu.VMEM((CHUNK, Hd), jnp.float32),
            pltpu.VMEM((CHUNK, Hd), jnp.float32),
            pltpu.VMEM((CHUNK, Hd), jnp.float32),
            pltpu.VMEM((CHUNK,), jnp.int32),
            pltpu.VMEM((CHUNK,), jnp.int32),
            pltpu.VMEM((CHUNK,), jnp.int32),
            pltpu.VMEM((CHUNK,), jnp.int32),
            pltpu.VMEM_SHARED((TR, Hd), jnp.float32),
        ] + [pltpu.SemaphoreType.DMA] * 6,
    )(src, dst, news_x, p, zeros_tile)

    c_blk = 1024
    n_blk = (C + c_blk - 1) // c_blk
    out = pl.pallas_call(
        _final_body,
        grid=(n_blk,),
        in_specs=[
            pl.BlockSpec((1, c_blk, Hd), lambda i: (0, i, 0)),
            pl.BlockSpec((1, c_blk, Hd), lambda i: (1, i, 0)),
            pl.BlockSpec((1, c_blk // Hd, Hd),
                         lambda i: (0, DEN0 // (c_blk // Hd) + i, 0)),
            pl.BlockSpec((1, c_blk // Hd, Hd),
                         lambda i: (1, DEN0 // (c_blk // Hd) + i, 0)),
        ],
        out_specs=pl.BlockSpec((c_blk, Hd), lambda i: (i, 0)),
        out_shape=jax.ShapeDtypeStruct((C, Hd), jnp.float32),
    )(parts, parts, parts, parts)
    return out


# speedup vs baseline: 9.6572x; 1.2604x over previous
import functools

import jax
import jax.numpy as jnp
from jax import lax
from jax.experimental import pallas as pl
from jax.experimental.pallas import tpu as pltpu
from jax.experimental.pallas import tpu_sc as plsc

NC, NS, L = 2, 16, 16
NW = NC * NS
H = 128
HB = H // L
CHUNK = 80

CP = 10240
DEN0 = CP
TR = 10368


def _proj_body(nx_ref, cx_ref, wn_ref, wc_ref, a_ref, b_ref):
    dn = (((1,), (1,)), ((), ()))
    a_ref[...] = lax.dot_general(nx_ref[...], wn_ref[...], dn,
                                 preferred_element_type=jnp.float32)
    b_ref[...] = lax.dot_general(cx_ref[...], wc_ref[...], dn,
                                 preferred_element_type=jnp.float32)


def _score_body(g_ref, v_ref, p_ref):
    t = jnp.tanh(g_ref[...])
    s = jnp.sum(t * v_ref[...], axis=1)
    p_ref[...] = jnp.exp(s)


def _final_body(f0_ref, f1_ref, f2_ref, f3_ref,
                d0_ref, d1_ref, d2_ref, d3_ref, o_ref):
    acc = f0_ref[0] + f1_ref[0] + f2_ref[0] + f3_ref[0]
    dpk = d0_ref[0] + d1_ref[0] + d2_ref[0] + d3_ref[0]
    n = acc.shape[0]
    rsel = (lax.broadcasted_iota(jnp.int32, (n, n // H), 0) // H
            == lax.broadcasted_iota(jnp.int32, (n, n // H), 1))
    den_rows = jax.lax.dot_general(rsel.astype(jnp.float32), dpk,
                                   (((1,), (0,)), ((), ())),
                                   preferred_element_type=jnp.float32)
    lsel = (lax.broadcasted_iota(jnp.int32, (n, H), 1)
            == lax.broadcasted_iota(jnp.int32, (n, H), 0) % H)
    den = jnp.sum(jnp.where(lsel, den_rows, 0.0), axis=1, keepdims=True)
    o_ref[...] = acc / jnp.maximum(den, 1e-9)


GC = 128


def _gather_add_body(src_hbm, dst_hbm, a_hbm, b_hbm, g_hbm,
                     si_v, di_v, g0_v, g1_v,
                     sa0, sa1, sb0, sb1, sw0, sw1):
    wid = lax.axis_index("s") * NC + lax.axis_index("c")
    epw = src_hbm.shape[0] // NW
    base = wid * epw
    n_full = epw // GC
    tail = epw - n_full * GC

    pltpu.sync_copy(src_hbm.at[pl.ds(base, epw)], si_v)
    pltpu.sync_copy(dst_hbm.at[pl.ds(base, epw)], di_v)

    def pair(k, _):
        c0 = 2 * k * GC
        c1 = c0 + GC
        a0 = pltpu.async_copy(a_hbm.at[si_v.at[pl.ds(c0, GC)]], g0_v, sa0)
        a1 = pltpu.async_copy(a_hbm.at[si_v.at[pl.ds(c1, GC)]], g1_v, sa1)
        a0.wait()
        b0 = pltpu.async_copy(b_hbm.at[di_v.at[pl.ds(c0, GC)]], g0_v, sb0,
                              add=True)
        a1.wait()
        b1 = pltpu.async_copy(b_hbm.at[di_v.at[pl.ds(c1, GC)]], g1_v, sb1,
                              add=True)
        b0.wait()
        w0 = pltpu.async_copy(g0_v, g_hbm.at[pl.ds(base + c0, GC)], sw0)
        b1.wait()
        w1 = pltpu.async_copy(g1_v, g_hbm.at[pl.ds(base + c1, GC)], sw1)
        w0.wait()
        w1.wait()
        return 0

    lax.fori_loop(0, n_full // 2, pair, 0)

    def one_chunk(off, size):
        pltpu.async_copy(a_hbm.at[si_v.at[pl.ds(off, size)]],
                         g0_v.at[pl.ds(0, size)], sa0).wait()
        pltpu.async_copy(b_hbm.at[di_v.at[pl.ds(off, size)]],
                         g0_v.at[pl.ds(0, size)], sb0, add=True).wait()
        pltpu.async_copy(g0_v.at[pl.ds(0, size)],
                         g_hbm.at[pl.ds(base + off, size)], sw0).wait()

    if n_full % 2:
        one_chunk((n_full - 1) * GC, GC)
    if tail:
        one_chunk(n_full * GC, tail)


SEC = 2000


def _accum_body(src_hbm, dst_hbm, x_hbm, p_hbm, z_hbm, parts_hbm,
                si_v, di_v, p_v, x0_v, x1_v, o0_v, o1_v,
                i0_v, i1_v, j0_v, j1_v, acc_sh,
                sg0, sg1, sf0, sf1, so0, so1):
    cid = lax.axis_index("c")
    sid = lax.axis_index("s")
    wid = sid * NC + cid
    epw = src_hbm.shape[0] // NW
    base = wid * epw
    rpt = acc_sh.shape[0] // NS

    pltpu.sync_copy(z_hbm, acc_sh.at[pl.ds(sid * rpt, rpt)])
    plsc.subcore_barrier()

    lane_iota = [lax.iota(jnp.int32, L) + h * L for h in range(HB)]

    def chunk_scatter(coff, x_v, o_v, i_v, j_v, sf, so):
        def group(k, _):
            eoff = coff + k * L
            ps = p_v[pl.ds(eoff, L)]
            di = di_v[pl.ds(eoff, L)]
            i_v[pl.ds(k * L, L)] = di
            j_v[pl.ds(k * L, L)] = DEN0 + lax.shift_right_logical(di, 7)
            dm = jnp.bitwise_and(di, H - 1)
            for j in range(L):
                e = k * L + j
                pb = lax.broadcast(ps[j], (L,))
                db = lax.broadcast(dm[j], (L,))
                for h in range(HB):
                    x_v[e, pl.ds(h * L, L)] = x_v[e, pl.ds(h * L, L)] * pb
                    o_v[e, pl.ds(h * L, L)] = jnp.where(lane_iota[h] == db,
                                                        pb, 0.0)
            return 0

        lax.fori_loop(0, CHUNK // L, group, 0)
        f = pltpu.async_copy(x_v, acc_sh.at[i_v], sf, add=True)
        d = pltpu.async_copy(o_v, acc_sh.at[j_v], so, add=True)
        return f, d

    def section(s, _):
        soff = base + s * SEC
        pltpu.sync_copy(src_hbm.at[pl.ds(soff, SEC)], si_v)
        pltpu.sync_copy(dst_hbm.at[pl.ds(soff, SEC)], di_v)
        pltpu.sync_copy(p_hbm.at[pl.ds(soff, SEC)], p_v)

        def pair(k, _):
            c0 = 2 * k * CHUNK
            c1 = c0 + CHUNK
            g0 = pltpu.async_copy(x_hbm.at[si_v.at[pl.ds(c0, CHUNK)]],
                                  x0_v, sg0)
            g1 = pltpu.async_copy(x_hbm.at[si_v.at[pl.ds(c1, CHUNK)]],
                                  x1_v, sg1)
            g0.wait()
            f0, d0 = chunk_scatter(c0, x0_v, o0_v, i0_v, j0_v, sf0, so0)
            g1.wait()
            f1, d1 = chunk_scatter(c1, x1_v, o1_v, i1_v, j1_v, sf1, so1)
            f0.wait()
            d0.wait()
            f1.wait()
            d1.wait()
            return 0

        n_chunks = SEC // CHUNK
        lax.fori_loop(0, n_chunks // 2, pair, 0)

        ct = (n_chunks - 1) * CHUNK
        pltpu.async_copy(x_hbm.at[si_v.at[pl.ds(ct, CHUNK)]],
                         x0_v, sg0).wait()
        ft, dt = chunk_scatter(ct, x0_v, o0_v, i0_v, j0_v, sf0, so0)
        ft.wait()
        dt.wait()
        return 0

    lax.fori_loop(0, epw // SEC, section, 0)

    plsc.subcore_barrier()
    pltpu.sync_copy(acc_sh.at[pl.ds(sid * rpt, rpt)],
                    parts_hbm.at[cid, pl.ds(sid * rpt, rpt)])


def kernel(news_x, company_x, edge_index, num_companies, W_news, W_company, v):
    N, Hd = news_x.shape
    C = company_x.shape[0]
    E = edge_index.shape[1]
    src = edge_index[0]
    dst = edge_index[1]

    rows_blk = 1000
    a, b = pl.pallas_call(
        _proj_body,
        grid=(N // rows_blk,),
        in_specs=[
            pl.BlockSpec((rows_blk, Hd), lambda i: (i, 0)),
            pl.BlockSpec((rows_blk, Hd), lambda i: (i, 0)),
            pl.BlockSpec((Hd, Hd), lambda i: (0, 0)),
            pl.BlockSpec((Hd, Hd), lambda i: (0, 0)),
        ],
        out_specs=[
            pl.BlockSpec((rows_blk, Hd), lambda i: (i, 0)),
            pl.BlockSpec((rows_blk, Hd), lambda i: (i, 0)),
        ],
        out_shape=[
            jax.ShapeDtypeStruct((N, Hd), jnp.float32),
            jax.ShapeDtypeStruct((C, Hd), jnp.float32),
        ],
    )(news_x, company_x, W_news, W_company)

    mesh = plsc.VectorSubcoreMesh(core_axis_name="c", subcore_axis_name="s",
                                  num_cores=NC, num_subcores=NS)

    def k2(src_b, dst_b):
        eb = src_b.shape[0]
        return pl.kernel(
            _gather_add_body,
            out_type=jax.ShapeDtypeStruct((eb, Hd), jnp.float32),
            mesh=mesh,
            scratch_types=[
                pltpu.VMEM((eb // NW,), jnp.int32),
                pltpu.VMEM((eb // NW,), jnp.int32),
                pltpu.VMEM((GC, Hd), jnp.float32),
                pltpu.VMEM((GC, Hd), jnp.float32),
            ] + [pltpu.SemaphoreType.DMA] * 6,
        )(src_b, dst_b, a, b)

    def k3(g_b):
        eb = g_b.shape[0]
        e_blk = 512
        return pl.pallas_call(
            _score_body,
            grid=(eb // e_blk,),
            in_specs=[
                pl.BlockSpec((e_blk, Hd), lambda i: (i, 0)),
                pl.BlockSpec((1, Hd), lambda i: (0, 0)),
            ],
            out_specs=pl.BlockSpec((e_blk,), lambda i: (i,)),
            out_shape=jax.ShapeDtypeStruct((eb,), jnp.float32),
        )(g_b, v)

    zeros_tile = jnp.zeros((TR // NS, Hd), jnp.float32)

    def k4(src_b, dst_b, p_b):
        return pl.kernel(
            _accum_body,
            out_type=jax.ShapeDtypeStruct((NC, TR, Hd), jnp.float32),
            mesh=mesh,
            scratch_types=[
                pltpu.VMEM((SEC,), jnp.int32),
                pltpu.VMEM((SEC,), jnp.int32),
                pltpu.VMEM((SEC,), jnp.float32),
                pltpu.VMEM((CHUNK, Hd), jnp.float32),
                pltpu.VMEM((CHUNK, Hd), jnp.float32),
                pltpu.VMEM((CHUNK, Hd), jnp.float32),
                pltpu.VMEM((CHUNK, Hd), jnp.float32),
                pltpu.VMEM((CHUNK,), jnp.int32),
                pltpu.VMEM((CHUNK,), jnp.int32),
                pltpu.VMEM((CHUNK,), jnp.int32),
                pltpu.VMEM((CHUNK,), jnp.int32),
                pltpu.VMEM_SHARED((TR, Hd), jnp.float32),
            ] + [pltpu.SemaphoreType.DMA] * 6,
        )(src_b, dst_b, news_x, p_b, zeros_tile)

    E0 = E * 3 // 5
    g0 = k2(src[:E0], dst[:E0])
    g1 = k2(src[E0:], dst[E0:])
    p0 = k3(g0)
    p1 = k3(g1)
    parts0 = k4(src[:E0], dst[:E0], p0)
    parts1 = k4(src[E0:], dst[E0:], p1)

    c_blk = 1024
    n_blk = (C + c_blk - 1) // c_blk
    fspec = [pl.BlockSpec((1, c_blk, Hd), lambda i, c=c: (c, i, 0))
             for c in (0, 1)]
    dspec = [pl.BlockSpec((1, c_blk // Hd, Hd),
                          lambda i, c=c: (c, DEN0 // (c_blk // Hd) + i, 0))
             for c in (0, 1)]
    out = pl.pallas_call(
        _final_body,
        grid=(n_blk,),
        in_specs=fspec + fspec + dspec + dspec,
        out_specs=pl.BlockSpec((c_blk, Hd), lambda i: (i, 0)),
        out_shape=jax.ShapeDtypeStruct((C, Hd), jnp.float32),
    )(parts0, parts0, parts1, parts1, parts0, parts0, parts1, parts1)
    return out
